# baseline (speedup 1.0000x reference)
import functools

import jax
import jax.numpy as jnp
from jax import lax
from jax.experimental import pallas as pl
from jax.experimental.pallas import tpu as pltpu
from jax.experimental.pallas import tpu_sc as plsc

_BATCH = 4096
_M = 20
_K = 30
_NC = 2
_NS = 16
_NW = _NC * _NS
_TOTAL = _BATCH * _M
_PER_W = _TOTAL // _NW
_C = 256
_NCHUNK = _PER_W // _C
_FLATW = 1000001 * _K
_VROWS = 234376


def _make_kernel():
  mesh = plsc.VectorSubcoreMesh(core_axis_name="c", subcore_axis_name="s")

  @functools.partial(
      pl.kernel,
      mesh=mesh,
      compiler_params=pltpu.CompilerParams(needs_layout_passes=False),
      out_type=jax.ShapeDtypeStruct((_TOTAL * _K,), jnp.int32),
      scratch_types=[
          pltpu.VMEM((_PER_W,), jnp.int32),
          pltpu.VMEM((4, 128), jnp.int32),
          pltpu.VMEM((_C,), jnp.int32),
          pltpu.VMEM((2 * _C, 128), jnp.int32),
          pltpu.VMEM((_C * _K,), jnp.int32),
          pltpu.SemaphoreType.DMA,
      ],
  )
  def gather_kernel(idx_hbm, table_hbm, out_hbm, idx_v, eidx_v, s_v,
                    win_v, cbuf_v, sem):
    wid = lax.axis_index("s") * _NC + lax.axis_index("c")
    base = wid * _PER_W
    pltpu.sync_copy(idx_hbm.at[pl.ds(base, _PER_W)], idx_v)
    lanes = lax.iota(jnp.int32, 16)

    def chunk_body(c, _):
      for b in range(_C // 16):
        v = idx_v[pl.ds(c * _C + b * 16, 16)]
        m = v * _K
        r = m >> 7
        s_v[pl.ds(b * 16, 16)] = m & 127
        pos = 32 * b + 2 * lanes
        plsc.store_scatter(eidx_v, [pos >> 7, pos & 127], r)
        pos1 = pos + 1
        plsc.store_scatter(eidx_v, [pos1 >> 7, pos1 & 127],
                           jnp.minimum(r + 1, _VROWS - 1))
      cps = []
      for j in range(4):
        cp = pltpu.make_async_copy(
            table_hbm.at[eidx_v.at[j]],
            win_v.at[pl.ds(j * 128, 128)], sem)
        cp.start()
        cps.append(cp)
      for cp in cps:
        cp.wait()
      for b in range(_C // 16):
        t = b * 16 + lanes
        srcbase = t * 256 + s_v[pl.ds(b * 16, 16)]
        dstbase = t * _K
        for j in range(_K):
          a = srcbase + j
          x = plsc.load_gather(win_v, [a >> 7, a & 127])
          plsc.store_scatter(cbuf_v, [dstbase + j], x)
      pltpu.sync_copy(
          cbuf_v, out_hbm.at[pl.ds((base + c * _C) * _K, _C * _K)])
      return 0

    lax.fori_loop(0, _NCHUNK, chunk_body, 0, unroll=False)

  return gather_kernel


_gather = _make_kernel()


@jax.jit
def kernel(alias_indices, alias2entity_table):
  flat = alias2entity_table.reshape(_FLATW)
  tpad = jnp.concatenate(
      [flat, jnp.zeros((_VROWS * 128 - _FLATW,), jnp.int32)])
  t128 = tpad.reshape(_VROWS, 128)
  idx = alias_indices.reshape(_TOTAL).astype(jnp.int32)
  out = _gather(idx, t128)
  return out.reshape(_BATCH, _M, _K)


# speedup vs baseline: 2.1232x; 2.1232x over previous
import functools

import jax
import jax.numpy as jnp
from jax import lax
from jax.experimental import pallas as pl
from jax.experimental.pallas import tpu as pltpu
from jax.experimental.pallas import tpu_sc as plsc

_BATCH = 4096
_M = 20
_K = 30
_NC = 2
_NS = 16
_NW = _NC * _NS
_TOTAL = _BATCH * _M
_PER_W = _TOTAL // _NW
_ROWS_W = _BATCH // _NW
_CB = 16
_C = _CB * _M
_NCHUNK = _ROWS_W // _CB
_NBLK = _C // 16


def _make_kernel():
  mesh = plsc.VectorSubcoreMesh(core_axis_name="c", subcore_axis_name="s")

  @functools.partial(
      pl.kernel,
      mesh=mesh,
      compiler_params=pltpu.CompilerParams(needs_layout_passes=False),
      out_type=jax.ShapeDtypeStruct((_BATCH, _M, _K), jnp.int32),
      scratch_types=[
          pltpu.VMEM((_PER_W,), jnp.int32),
          pltpu.VMEM((_CB, _M, _K), jnp.int32),
          pltpu.SemaphoreType.DMA,
      ],
  )
  def gather_kernel(idx_hbm, table_hbm, out_hbm, idx_v, rows_v, sem):
    wid = lax.axis_index("s") * _NC + lax.axis_index("c")
    base = wid * _PER_W
    pltpu.sync_copy(idx_hbm.at[pl.ds(base, _PER_W)], idx_v)

    def chunk_body(c, _):
      cps = []
      for b in range(_NBLK):
        x = idx_v[pl.ds(c * _C + 16 * b, 16)]
        for l in range(16):
          t = 16 * b + l
          cp = pltpu.make_async_copy(
              table_hbm.at[x[l]], rows_v.at[t // _M, t % _M], sem)
          cp.start()
          cps.append(cp)
        if b >= 2:
          for cp in cps[16 * (b - 2):16 * (b - 1)]:
            cp.wait()
      for cp in cps[16 * (_NBLK - 2):]:
        cp.wait()
      pltpu.sync_copy(
          rows_v, out_hbm.at[pl.ds(wid * _ROWS_W + c * _CB, _CB)])
      return 0

    lax.fori_loop(0, _NCHUNK, chunk_body, 0, unroll=False)

  return gather_kernel


_gather = _make_kernel()


@jax.jit
def kernel(alias_indices, alias2entity_table):
  idx = alias_indices.reshape(_TOTAL).astype(jnp.int32)
  return _gather(idx, alias2entity_table)


# speedup vs baseline: 2.1241x; 1.0004x over previous
import functools

import jax
import jax.numpy as jnp
from jax import lax
from jax.experimental import pallas as pl
from jax.experimental.pallas import tpu as pltpu
from jax.experimental.pallas import tpu_sc as plsc

_BATCH = 4096
_M = 20
_K = 30
_NC = 2
_NS = 16
_NW = _NC * _NS
_TOTAL = _BATCH * _M
_PER_W = _TOTAL // _NW
_ROWS_W = _BATCH // _NW
_CB = 16
_C = _CB * _M
_NCHUNK = _ROWS_W // _CB
_NBLK = _C // 16


def _make_kernel():
  mesh = plsc.VectorSubcoreMesh(core_axis_name="c", subcore_axis_name="s")

  @functools.partial(
      pl.kernel,
      mesh=mesh,
      compiler_params=pltpu.CompilerParams(
          needs_layout_passes=False, use_tc_tiling_on_sc=True),
      out_type=jax.ShapeDtypeStruct((_BATCH, _M, _K), jnp.int32),
      scratch_types=[
          pltpu.VMEM((_PER_W,), jnp.int32),
          pltpu.VMEM((_CB, _M, _K), jnp.int32),
          pltpu.SemaphoreType.DMA,
      ],
  )
  def gather_kernel(idx_hbm, table_hbm, out_hbm, idx_v, rows_v, sem):
    wid = lax.axis_index("s") * _NC + lax.axis_index("c")
    base = wid * _PER_W
    pltpu.sync_copy(idx_hbm.at[pl.ds(base, _PER_W)], idx_v)

    def chunk_body(c, _):
      cps = []
      for b in range(_NBLK):
        x = idx_v[pl.ds(c * _C + 16 * b, 16)]
        for l in range(16):
          t = 16 * b + l
          cp = pltpu.make_async_copy(
              table_hbm.at[x[l]], rows_v.at[t // _M, t % _M], sem)
          cp.start()
          cps.append(cp)
        if b >= 2:
          for cp in cps[16 * (b - 2):16 * (b - 1)]:
            cp.wait()
      for cp in cps[16 * (_NBLK - 2):]:
        cp.wait()
      pltpu.sync_copy(
          rows_v, out_hbm.at[pl.ds(wid * _ROWS_W + c * _CB, _CB)])
      return 0

    lax.fori_loop(0, _NCHUNK, chunk_body, 0, unroll=False)

  return gather_kernel


_gather = _make_kernel()


@jax.jit
def kernel(alias_indices, alias2entity_table):
  idx = alias_indices.reshape(_TOTAL).astype(jnp.int32)
  return _gather(idx, alias2entity_table)


# speedup vs baseline: 2.1275x; 1.0016x over previous
import functools

import jax
import jax.numpy as jnp
from jax import lax
from jax.experimental import pallas as pl
from jax.experimental.pallas import tpu as pltpu
from jax.experimental.pallas import tpu_sc as plsc

_BATCH = 4096
_M = 20
_K = 30
_NC = 2
_NS = 16
_NW = _NC * _NS
_TOTAL = _BATCH * _M
_PER_W = _TOTAL // _NW
_ROWS_W = _BATCH // _NW
_CB = 16
_C = _CB * _M
_NCHUNK = _ROWS_W // _CB
_NBLK = _C // 16


def _make_kernel():
  mesh = plsc.VectorSubcoreMesh(core_axis_name="c", subcore_axis_name="s")

  @functools.partial(
      pl.kernel,
      mesh=mesh,
      compiler_params=pltpu.CompilerParams(use_tc_tiling_on_sc=True),
      out_type=jax.ShapeDtypeStruct((_BATCH, _M, _K), jnp.int32),
      scratch_types=[
          pltpu.VMEM((_PER_W,), jnp.int32),
          pltpu.VMEM((_CB, _M, _K), jnp.int32),
          pltpu.SemaphoreType.DMA,
      ],
  )
  def gather_kernel(idx_hbm, table_hbm, out_hbm, idx_v, rows_v, sem):
    wid = lax.axis_index("s") * _NC + lax.axis_index("c")
    base = wid * _PER_W
    pltpu.sync_copy(idx_hbm.at[pl.ds(base, _PER_W)], idx_v)

    def chunk_body(c, _):
      cps = []
      for b in range(_NBLK):
        x = idx_v[pl.ds(c * _C + 16 * b, 16)]
        for l in range(16):
          t = 16 * b + l
          cp = pltpu.make_async_copy(
              table_hbm.at[x[l]], rows_v.at[t // _M, t % _M], sem)
          cp.start()
          cps.append(cp)
        if b >= 2:
          for cp in cps[16 * (b - 2):16 * (b - 1)]:
            cp.wait()
      for cp in cps[16 * (_NBLK - 2):]:
        cp.wait()
      pltpu.sync_copy(
          rows_v, out_hbm.at[pl.ds(wid * _ROWS_W + c * _CB, _CB)])
      return 0

    lax.fori_loop(0, _NCHUNK, chunk_body, 0, unroll=False)

  return gather_kernel


_gather = _make_kernel()


@jax.jit
def kernel(alias_indices, alias2entity_table):
  idx = alias_indices.reshape(_TOTAL).astype(jnp.int32)
  return _gather(idx, alias2entity_table)
